# baseline (speedup 1.0000x reference)
import functools

import jax
import jax.numpy as jnp
from jax import lax
from jax.experimental import pallas as pl
from jax.experimental.pallas import tpu as pltpu
from jax.experimental.pallas import tpu_sc as plsc

VOCAB = 21128
DIM = 768
SEQ = 1000
PAD_ID = 0

_NUM_WORKERS = 32
_ROWS_PER_WORKER = 32
_LAST_BASE = SEQ - _ROWS_PER_WORKER
_CHUNK = 8
_NCHUNK = _ROWS_PER_WORKER // _CHUNK


def _gather_body(tok_hbm, table_hbm, out_hbm,
                 idx_v, buf0, buf1, gsem0, gsem1, wsem0, wsem1):
    wid = lax.axis_index("s") * 2 + lax.axis_index("c")
    base = jnp.minimum(wid * _ROWS_PER_WORKER, _LAST_BASE)
    for c in range(_NCHUNK):
        pltpu.sync_copy(
            tok_hbm.at[pl.ds(base + c * _CHUNK, _CHUNK)], idx_v.at[c]
        )

    bufs = [buf0, buf1]
    gsems = [gsem0, gsem1]
    wsems = [wsem0, wsem1]

    def gather(c):
        p = c % 2
        return pltpu.make_async_copy(
            table_hbm.at[idx_v.at[c]],
            bufs[p], gsems[p],
        )

    def writeback(c):
        p = c % 2
        return pltpu.make_async_copy(
            bufs[p], out_hbm.at[pl.ds(base + c * _CHUNK, _CHUNK)], wsems[p],
        )

    gather(0).start()
    for c in range(_NCHUNK):
        gather(c).wait()
        writeback(c).start()
        if c + 1 < _NCHUNK:
            if c >= 1:
                writeback(c - 1).wait()
            gather(c + 1).start()
    writeback(_NCHUNK - 2).wait()
    writeback(_NCHUNK - 1).wait()


@jax.jit
def _embed(tokenids, table):
    mesh = plsc.VectorSubcoreMesh(core_axis_name="c", subcore_axis_name="s")
    run = pl.kernel(
        _gather_body,
        out_type=jax.ShapeDtypeStruct((SEQ, DIM), jnp.float32),
        mesh=mesh,
        scratch_types=[
            pltpu.VMEM((_NCHUNK, _CHUNK), jnp.int32),
            pltpu.VMEM((_CHUNK, DIM), jnp.float32),
            pltpu.VMEM((_CHUNK, DIM), jnp.float32),
            pltpu.SemaphoreType.DMA,
            pltpu.SemaphoreType.DMA,
            pltpu.SemaphoreType.DMA,
            pltpu.SemaphoreType.DMA,
        ],
    )
    return run(tokenids, table)


def kernel(tokenids, table):
    token_ebd = _embed(tokenids, table)
    pad = jnp.array([PAD_ID], dtype=tokenids.dtype)
    labels = jnp.concatenate((tokenids[1:], pad))
    return (token_ebd, labels)


# speedup vs baseline: 1.0937x; 1.0937x over previous
import functools

import jax
import jax.numpy as jnp
from jax import lax
from jax.experimental import pallas as pl
from jax.experimental.pallas import tpu as pltpu
from jax.experimental.pallas import tpu_sc as plsc

VOCAB = 21128
DIM = 768
SEQ = 1000
PAD_ID = 0

_NUM_WORKERS = 32
_ROWS_PER_WORKER = 32
_LAST_BASE = SEQ - _ROWS_PER_WORKER
_CHUNK = 16
_NCHUNK = _ROWS_PER_WORKER // _CHUNK


def _gather_body(tok_hbm, table_hbm, out_hbm,
                 idx_v, buf0, buf1, gsem0, gsem1, wsem0, wsem1):
    wid = lax.axis_index("s") * 2 + lax.axis_index("c")
    base = jnp.minimum(wid * _ROWS_PER_WORKER, _LAST_BASE)
    for c in range(_NCHUNK):
        pltpu.sync_copy(
            tok_hbm.at[pl.ds(base + c * _CHUNK, _CHUNK)], idx_v.at[c]
        )

    bufs = [buf0, buf1]
    gsems = [gsem0, gsem1]
    wsems = [wsem0, wsem1]

    def gather(c):
        p = c % 2
        return pltpu.make_async_copy(
            table_hbm.at[idx_v.at[c]],
            bufs[p], gsems[p],
        )

    def writeback(c):
        p = c % 2
        return pltpu.make_async_copy(
            bufs[p], out_hbm.at[pl.ds(base + c * _CHUNK, _CHUNK)], wsems[p],
        )

    gather(0).start()
    for c in range(_NCHUNK):
        gather(c).wait()
        writeback(c).start()
        if c + 1 < _NCHUNK:
            if c >= 1:
                writeback(c - 1).wait()
            gather(c + 1).start()
    writeback(_NCHUNK - 2).wait()
    writeback(_NCHUNK - 1).wait()


@jax.jit
def _embed(tokenids, table):
    mesh = plsc.VectorSubcoreMesh(core_axis_name="c", subcore_axis_name="s")
    run = pl.kernel(
        _gather_body,
        out_type=jax.ShapeDtypeStruct((SEQ, DIM), jnp.float32),
        mesh=mesh,
        scratch_types=[
            pltpu.VMEM((_NCHUNK, _CHUNK), jnp.int32),
            pltpu.VMEM((_CHUNK, DIM), jnp.float32),
            pltpu.VMEM((_CHUNK, DIM), jnp.float32),
            pltpu.SemaphoreType.DMA,
            pltpu.SemaphoreType.DMA,
            pltpu.SemaphoreType.DMA,
            pltpu.SemaphoreType.DMA,
        ],
    )
    return run(tokenids, table)


def kernel(tokenids, table):
    token_ebd = _embed(tokenids, table)
    pad = jnp.array([PAD_ID], dtype=tokenids.dtype)
    labels = jnp.concatenate((tokenids[1:], pad))
    return (token_ebd, labels)


# speedup vs baseline: 1.1373x; 1.0398x over previous
import functools

import jax
import jax.numpy as jnp
from jax import lax
from jax.experimental import pallas as pl
from jax.experimental.pallas import tpu as pltpu
from jax.experimental.pallas import tpu_sc as plsc

VOCAB = 21128
DIM = 768
SEQ = 1000
PAD_ID = 0

_NUM_WORKERS = 32
_ROWS_PER_WORKER = 32
_LAST_BASE = SEQ - _ROWS_PER_WORKER


def _gather_body(tok_hbm, table_hbm, out_hbm, idx_v, rows_v, sem):
    wid = lax.axis_index("s") * 2 + lax.axis_index("c")
    base = jnp.minimum(wid * _ROWS_PER_WORKER, _LAST_BASE)
    pltpu.sync_copy(tok_hbm.at[pl.ds(base, _ROWS_PER_WORKER)], idx_v)
    pltpu.async_copy(table_hbm.at[idx_v], rows_v, sem).wait()
    pltpu.sync_copy(rows_v, out_hbm.at[pl.ds(base, _ROWS_PER_WORKER)])


@jax.jit
def _embed(tokenids, table):
    mesh = plsc.VectorSubcoreMesh(core_axis_name="c", subcore_axis_name="s")
    run = pl.kernel(
        _gather_body,
        out_type=jax.ShapeDtypeStruct((SEQ, DIM), jnp.float32),
        mesh=mesh,
        scratch_types=[
            pltpu.VMEM((_ROWS_PER_WORKER,), jnp.int32),
            pltpu.VMEM((_ROWS_PER_WORKER, DIM), jnp.float32),
            pltpu.SemaphoreType.DMA,
        ],
    )
    return run(tokenids, table)


def kernel(tokenids, table):
    token_ebd = _embed(tokenids, table)
    pad = jnp.array([PAD_ID], dtype=tokenids.dtype)
    labels = jnp.concatenate((tokenids[1:], pad))
    return (token_ebd, labels)


# speedup vs baseline: 1.1399x; 1.0023x over previous
import functools

import jax
import jax.numpy as jnp
from jax import lax
from jax.experimental import pallas as pl
from jax.experimental.pallas import tpu as pltpu
from jax.experimental.pallas import tpu_sc as plsc

VOCAB = 21128
DIM = 768
SEQ = 1000
PAD_ID = 0

_NUM_WORKERS = 32
_ROWS_PER_WORKER = 32
_LAST_BASE = SEQ - _ROWS_PER_WORKER


def _gather_body(tok_hbm, table_hbm, out_hbm, idx_v, rows_v, sem):
    wid = lax.axis_index("c") * 16 + lax.axis_index("s")
    base = jnp.minimum(wid * _ROWS_PER_WORKER, _LAST_BASE)
    pltpu.sync_copy(tok_hbm.at[pl.ds(base, _ROWS_PER_WORKER)], idx_v)
    pltpu.async_copy(table_hbm.at[idx_v], rows_v, sem).wait()
    pltpu.sync_copy(rows_v, out_hbm.at[pl.ds(base, _ROWS_PER_WORKER)])


@jax.jit
def _embed(tokenids, table):
    mesh = plsc.VectorSubcoreMesh(core_axis_name="c", subcore_axis_name="s")
    run = pl.kernel(
        _gather_body,
        out_type=jax.ShapeDtypeStruct((SEQ, DIM), jnp.float32),
        mesh=mesh,
        scratch_types=[
            pltpu.VMEM((_ROWS_PER_WORKER,), jnp.int32),
            pltpu.VMEM((_ROWS_PER_WORKER, DIM), jnp.float32),
            pltpu.SemaphoreType.DMA,
        ],
    )
    return run(tokenids, table)


def kernel(tokenids, table):
    token_ebd = _embed(tokenids, table)
    pad = jnp.array([PAD_ID], dtype=tokenids.dtype)
    labels = jnp.concatenate((tokenids[1:], pad))
    return (token_ebd, labels)


# speedup vs baseline: 1.1439x; 1.0035x over previous
import jax
import jax.numpy as jnp
from jax import lax
from jax.experimental import pallas as pl
from jax.experimental.pallas import tpu as pltpu
from jax.experimental.pallas import tpu_sc as plsc

VOCAB = 21128
DIM = 768
SEQ = 1000
PAD_ID = 0

_NUM_WORKERS = 32
_ROWS_PER_WORKER = 32
_LAST_BASE = SEQ - _ROWS_PER_WORKER


def _gather_body(tok_hbm, table_hbm, out_hbm, idx_v, rows_v, sem):
    wid = lax.axis_index("c") * 16 + lax.axis_index("s")
    base = jnp.minimum(wid * _ROWS_PER_WORKER, _LAST_BASE)
    pltpu.sync_copy(tok_hbm.at[pl.ds(base, _ROWS_PER_WORKER)], idx_v)
    pltpu.async_copy(table_hbm.at[idx_v], rows_v, sem).wait()
    pltpu.sync_copy(rows_v, out_hbm.at[pl.ds(base, _ROWS_PER_WORKER)])


@jax.jit
def _embed(tokenids, table):
    mesh = plsc.VectorSubcoreMesh(core_axis_name="c", subcore_axis_name="s")
    run = pl.kernel(
        _gather_body,
        out_type=jax.ShapeDtypeStruct((SEQ, DIM), jnp.float32),
        mesh=mesh,
        scratch_types=[
            pltpu.VMEM((_ROWS_PER_WORKER,), jnp.int32),
            pltpu.VMEM((_ROWS_PER_WORKER, DIM), jnp.float32),
            pltpu.SemaphoreType.DMA,
        ],
    )
    return run(tokenids, table)


def kernel(tokenids, table):
    token_ebd = _embed(tokenids, table)
    pad = jnp.array([PAD_ID], dtype=tokenids.dtype)
    labels = jnp.concatenate((tokenids[1:], pad))
    return (token_ebd, labels)
